# baseline (speedup 1.0000x reference)
import jax
import jax.numpy as jnp
from jax.experimental import pallas as pl
from jax.experimental.pallas import tpu as pltpu

B = 64
S = 128
HIDDEN = 768
G = 12

_NCHUNK = 8
_ROWS = B * S * G
_CR = _ROWS // _NCHUNK


def _copy_body(x_ref, o_ref, sem):
    cps = [
        pltpu.make_async_copy(
            x_ref.at[pl.ds(i * _CR, _CR), :],
            o_ref.at[pl.ds(i * _CR, _CR), :],
            sem.at[i],
        )
        for i in range(_NCHUNK)
    ]
    for c in cps:
        c.start()
    for c in cps:
        c.wait()


def _copy(flat):
    return pl.pallas_call(
        _copy_body,
        in_specs=[pl.BlockSpec(memory_space=pltpu.MemorySpace.HBM)],
        out_specs=pl.BlockSpec(memory_space=pltpu.MemorySpace.HBM),
        out_shape=jax.ShapeDtypeStruct((_ROWS, HIDDEN), jnp.float32),
        scratch_shapes=[pltpu.SemaphoreType.DMA((_NCHUNK,))],
    )(flat)


def kernel(input_data_seq, batch_head_matrix, W1, b1, W2, b2):
    flat = batch_head_matrix.reshape(_ROWS, HIDDEN)
    out = _copy(flat)
    prob = jnp.zeros((B, G), jnp.float32)
    pm = jnp.zeros((B, S, HIDDEN), jnp.float32)
    return (prob, pm, out.reshape(B, S, G, HIDDEN))


# speedup vs baseline: 7.8444x; 7.8444x over previous
import functools
import math

import jax
import jax.numpy as jnp
from jax import lax
from jax.experimental import pallas as pl
from jax.experimental.pallas import tpu as pltpu

B = 64
S = 128
HIDDEN = 768
G = 12
SCALE = 12.0 / 11.0

CR = 384
SR = CR // G
N = (B * S * G) // CR
NBUF = 12
KPF = 8

XCH = 512
XN = (B * S) // XCH
XBUF = 4


def _body(xseq_ref, bhm_ref, w1_ref, b1_ref, w2_ref, b2_ref, gum_ref,
          prob_ref, pm_ref, copy_ref,
          inbuf, pmbuf, xbuf, meanbuf, idxv_ref,
          in_sem, co_sem, pm_sem, x_sem):

    def in_cp(i, slot):
        return pltpu.make_async_copy(
            bhm_ref.at[pl.ds(i * CR, CR), :], inbuf.at[slot], in_sem.at[slot])

    def co_cp(i, slot):
        return pltpu.make_async_copy(
            inbuf.at[slot], copy_ref.at[pl.ds(i * CR, CR), :], co_sem.at[slot])

    def pm_cp(i, slot):
        return pltpu.make_async_copy(
            pmbuf.at[slot], pm_ref.at[pl.ds(i * SR, SR), :], pm_sem.at[slot])

    def x_cp(c, slot):
        return pltpu.make_async_copy(
            xseq_ref.at[pl.ds(c * XCH, XCH), :], xbuf.at[slot], x_sem.at[slot])

    for i in range(KPF):
        in_cp(i, i).start()

    for c in range(XBUF):
        x_cp(c, c).start()
    for c in range(XN):
        x_cp(c, c % XBUF).wait()
        v = xbuf[c % XBUF].reshape(XCH // S, S, HIDDEN)
        meanbuf[pl.ds(c * (XCH // S), XCH // S), :] = (
            jnp.sum(v, axis=1) * (1.0 / S))
        if c + XBUF < XN:
            x_cp(c + XBUF, c % XBUF).start()

    mean = meanbuf[...]
    w1 = w1_ref[...]
    h1 = lax.dot_general(mean, w1, (((1,), (1,)), ((), ())),
                         precision=lax.Precision.HIGHEST,
                         preferred_element_type=jnp.float32) + b1_ref[...]
    a1 = 0.5 * h1 * (1.0 + lax.erf(h1 * (1.0 / math.sqrt(2.0))))
    w2 = w2_ref[...]
    h2 = lax.dot_general(a1, w2, (((1,), (1,)), ((), ())),
                         precision=lax.Precision.HIGHEST,
                         preferred_element_type=jnp.float32) + b2_ref[...]
    m = jnp.max(h2, axis=-1, keepdims=True)
    e = jnp.exp(h2 - m)
    prob = e / jnp.sum(e, axis=-1, keepdims=True)
    prob_ref[...] = prob
    scores = jnp.log(prob) + gum_ref[...]
    idx = jnp.argmax(scores, axis=-1).astype(jnp.int32)
    idxv_ref[...] = jnp.broadcast_to(idx[:, None], (B, CR))

    rows_c = lax.broadcasted_iota(jnp.int32, (SR, CR), 0)
    cols_c = lax.broadcasted_iota(jnp.int32, (SR, CR), 1)
    colg_c = cols_c - (cols_c // G) * G
    const_m = (cols_c // G == rows_c)

    def loop(i, carry):
        slot = i % NBUF
        in_cp(i, slot).wait()
        co_cp(i, slot).start()

        @pl.when(i >= NBUF)
        def _():
            pm_cp(i - NBUF, slot).wait()

        b = i // (N // B)
        idxrow = idxv_ref[pl.ds(b, 1), :]
        mb = jnp.where(const_m & (colg_c != idxrow), SCALE, 0.0)
        x = inbuf[slot]
        acc = lax.dot_general(mb, x, (((1,), (0,)), ((), ())),
                              precision=lax.Precision.HIGHEST,
                              preferred_element_type=jnp.float32)
        pmbuf[slot] = acc
        pm_cp(i, slot).start()

        j = i + KPF

        @pl.when(j < N)
        def _():
            sj = j % NBUF

            @pl.when(j >= NBUF)
            def _():
                co_cp(j - NBUF, sj).wait()

            in_cp(j, sj).start()

        return carry

    lax.fori_loop(0, N, loop, 0)

    for t in range(N - NBUF, N):
        co_cp(t, t % NBUF).wait()
    for t in range(N - NBUF, N):
        pm_cp(t, t % NBUF).wait()


def kernel(input_data_seq, batch_head_matrix, W1, b1, W2, b2):
    gumbel = jax.random.gumbel(jax.random.key(42), (B, G), jnp.float32)
    xseq = input_data_seq.reshape(B * S, HIDDEN)
    bhm_flat = batch_head_matrix.reshape(B * S * G, HIDDEN)
    hbm = pltpu.MemorySpace.HBM
    vmem = pltpu.MemorySpace.VMEM
    prob, pm, copy = pl.pallas_call(
        _body,
        in_specs=[
            pl.BlockSpec(memory_space=hbm),
            pl.BlockSpec(memory_space=hbm),
            pl.BlockSpec(memory_space=vmem),
            pl.BlockSpec(memory_space=vmem),
            pl.BlockSpec(memory_space=vmem),
            pl.BlockSpec(memory_space=vmem),
            pl.BlockSpec(memory_space=vmem),
        ],
        out_specs=[
            pl.BlockSpec(memory_space=vmem),
            pl.BlockSpec(memory_space=hbm),
            pl.BlockSpec(memory_space=hbm),
        ],
        out_shape=[
            jax.ShapeDtypeStruct((B, G), jnp.float32),
            jax.ShapeDtypeStruct((B * S, HIDDEN), jnp.float32),
            jax.ShapeDtypeStruct((B * S * G, HIDDEN), jnp.float32),
        ],
        scratch_shapes=[
            pltpu.VMEM((NBUF, CR, HIDDEN), jnp.float32),
            pltpu.VMEM((NBUF, SR, HIDDEN), jnp.float32),
            pltpu.VMEM((XBUF, XCH, HIDDEN), jnp.float32),
            pltpu.VMEM((B, HIDDEN), jnp.float32),
            pltpu.VMEM((B, CR), jnp.int32),
            pltpu.SemaphoreType.DMA((NBUF,)),
            pltpu.SemaphoreType.DMA((NBUF,)),
            pltpu.SemaphoreType.DMA((NBUF,)),
            pltpu.SemaphoreType.DMA((XBUF,)),
        ],
    )(xseq, bhm_flat, W1, b1.reshape(1, G), W2, b2.reshape(1, G), gumbel)
    return (prob, pm.reshape(B, S, HIDDEN), copy.reshape(B, S, G, HIDDEN))
